# baseline (speedup 1.0000x reference)
import functools

import jax
import jax.numpy as jnp
from jax import lax
from jax.experimental import pallas as pl
from jax.experimental.pallas import tpu as pltpu
from jax.experimental.pallas import tpu_sc as plsc

_K = 8192
_D = 32
_BETA = 0.25
_T = 256


_KBLK = 2048
_NK = _K // _KBLK


def _dist_argmin_body(lhs_ref, cb_ref, z2_ref, c2_ref, idx_ref, loss_ref):
    i = pl.program_id(0)

    iota = lax.broadcasted_iota(jnp.int32, (_T, _KBLK), 1)
    best_v = None
    best_i = None
    for k in range(_NK):
        mm = lax.dot_general(
            lhs_ref[...], cb_ref[k * _KBLK:(k + 1) * _KBLK, :],
            dimension_numbers=(((1,), (1,)), ((), ())),
            preferred_element_type=jnp.float32,
        )
        dist = (z2_ref[...] + c2_ref[:, k * _KBLK:(k + 1) * _KBLK]) - mm
        local_min = jnp.min(dist, axis=1, keepdims=True)
        masked = jnp.where(dist == local_min, iota, jnp.int32(2**31 - 1))
        local_idx = jnp.min(masked, axis=1, keepdims=True) + k * _KBLK
        if k == 0:
            best_v, best_i = local_min, local_idx
        else:
            upd = local_min < best_v
            best_v = jnp.where(upd, local_min, best_v)
            best_i = jnp.where(upd, local_idx, best_i)
        if k < _NK - 1:
            best_v = best_v.astype(jnp.bfloat16).astype(jnp.float32)

    idx_ref[...] = best_i
    partial = jnp.sum(best_v) * (_BETA / (_K * _D))

    @pl.when(i == 0)
    def _():
        loss_ref[...] = jnp.reshape(partial, (1, 1))

    @pl.when(i > 0)
    def _():
        loss_ref[...] = loss_ref[...] + jnp.reshape(partial, (1, 1))


def _dist_argmin(lhs_bf, c_t, z2, c2):
    n = lhs_bf.shape[0]
    grid = (n // _T,)
    return pl.pallas_call(
        _dist_argmin_body,
        grid=grid,
        in_specs=[
            pl.BlockSpec((_T, _D), lambda i: (i, 0)),
            pl.BlockSpec((_K, _D), lambda i: (0, 0)),
            pl.BlockSpec((_T, 1), lambda i: (i, 0)),
            pl.BlockSpec((1, _K), lambda i: (0, 0)),
        ],
        out_specs=[
            pl.BlockSpec((_T, 1), lambda i: (i, 0)),
            pl.BlockSpec((1, 1), lambda i: (0, 0)),
        ],
        out_shape=[
            jax.ShapeDtypeStruct((n, 1), jnp.int32),
            jax.ShapeDtypeStruct((1, 1), jnp.float32),
        ],
    )(lhs_bf, c_t, z2, c2)


def _sc_gather(table, idx):
    info = plsc.get_sparse_core_info()
    nc, ns = info.num_cores, info.num_subcores
    nw = nc * ns
    n = idx.shape[0]
    per_w = n // nw
    dp = 128
    table_p = jnp.pad(table, ((0, 0), (0, dp - _D)))
    mesh = plsc.VectorSubcoreMesh(core_axis_name="c", subcore_axis_name="s")

    @functools.partial(
        pl.kernel, mesh=mesh,
        out_type=jax.ShapeDtypeStruct((n, dp), jnp.float32),
        scratch_types=[
            pltpu.VMEM((per_w,), jnp.int32),
            pltpu.VMEM((per_w, dp), jnp.float32),
            pltpu.SemaphoreType.DMA,
        ],
    )
    def k(table_hbm, idx_hbm, out_hbm, idx_v, rows_v, sem):
        wid = lax.axis_index("s") * nc + lax.axis_index("c")
        base = wid * per_w
        pltpu.sync_copy(idx_hbm.at[pl.ds(base, per_w)], idx_v)
        pltpu.async_copy(table_hbm.at[idx_v], rows_v, sem).wait()
        pltpu.sync_copy(rows_v, out_hbm.at[pl.ds(base, per_w)])

    return k(table_p, idx)[:, :_D]


def kernel(z, codebook):
    b, d, h, w = z.shape
    z_flat = jnp.transpose(z, (0, 2, 3, 1)).reshape(-1, d)
    z2 = jnp.sum(z_flat ** 2, axis=1, keepdims=True)
    c2 = jnp.sum(codebook ** 2, axis=1)[None, :]
    cb_bf = codebook.astype(jnp.bfloat16)
    lhs_bf = (2.0 * z_flat).astype(jnp.bfloat16)

    idx2d, loss11 = _dist_argmin(lhs_bf, cb_bf, z2, c2)
    indices = idx2d[:, 0]

    zq_flat = _sc_gather(codebook, indices)
    z_q = jnp.transpose(zq_flat.reshape(b, h, w, d), (0, 3, 1, 2))
    z_q = z + lax.stop_gradient(z_q - z)
    return (z_q, indices.reshape(b, h * w), loss11[0, 0])


# speedup vs baseline: 1.0175x; 1.0175x over previous
import functools

import jax
import jax.numpy as jnp
from jax import lax
from jax.experimental import pallas as pl
from jax.experimental.pallas import tpu as pltpu
from jax.experimental.pallas import tpu_sc as plsc

_K = 8192
_D = 32
_BETA = 0.25
_T = 256


_KBLK = 2048
_NK = _K // _KBLK


def _dist_argmin_body(lhs_ref, cb_ref, z2_ref, c2_ref, idx_ref, loss_ref):
    i = pl.program_id(0)

    iota = lax.broadcasted_iota(jnp.int32, (_T, _KBLK), 1)
    best_v = None
    best_i = None
    for k in range(_NK):
        mm = lax.dot_general(
            lhs_ref[...], cb_ref[:, k * _KBLK:(k + 1) * _KBLK],
            dimension_numbers=(((1,), (0,)), ((), ())),
            preferred_element_type=jnp.float32,
        )
        dist = (z2_ref[...] + c2_ref[:, k * _KBLK:(k + 1) * _KBLK]) - mm
        local_min = jnp.min(dist, axis=1, keepdims=True)
        masked = jnp.where(dist == local_min, iota, jnp.int32(2**31 - 1))
        local_idx = jnp.min(masked, axis=1, keepdims=True) + k * _KBLK
        if k == 0:
            best_v, best_i = local_min, local_idx
        else:
            upd = local_min < best_v
            best_v = jnp.where(upd, local_min, best_v)
            best_i = jnp.where(upd, local_idx, best_i)
        if k < _NK - 1:
            best_v = best_v.astype(jnp.bfloat16).astype(jnp.float32)

    idx_ref[...] = best_i
    partial = jnp.sum(best_v) * (_BETA / (_K * _D))

    @pl.when(i == 0)
    def _():
        loss_ref[...] = jnp.reshape(partial, (1, 1))

    @pl.when(i > 0)
    def _():
        loss_ref[...] = loss_ref[...] + jnp.reshape(partial, (1, 1))


def _dist_argmin(lhs_bf, c_t, z2, c2):
    n = lhs_bf.shape[0]
    grid = (n // _T,)
    return pl.pallas_call(
        _dist_argmin_body,
        grid=grid,
        in_specs=[
            pl.BlockSpec((_T, _D), lambda i: (i, 0)),
            pl.BlockSpec((_D, _K), lambda i: (0, 0)),
            pl.BlockSpec((_T, 1), lambda i: (i, 0)),
            pl.BlockSpec((1, _K), lambda i: (0, 0)),
        ],
        out_specs=[
            pl.BlockSpec((_T, 1), lambda i: (i, 0)),
            pl.BlockSpec((1, 1), lambda i: (0, 0)),
        ],
        out_shape=[
            jax.ShapeDtypeStruct((n, 1), jnp.int32),
            jax.ShapeDtypeStruct((1, 1), jnp.float32),
        ],
    )(lhs_bf, c_t, z2, c2)


def _sc_gather(table, idx):
    info = plsc.get_sparse_core_info()
    nc, ns = info.num_cores, info.num_subcores
    nw = nc * ns
    n = idx.shape[0]
    per_w = n // nw
    dp = 128
    table_p = jnp.pad(table, ((0, 0), (0, dp - _D)))
    mesh = plsc.VectorSubcoreMesh(core_axis_name="c", subcore_axis_name="s")

    @functools.partial(
        pl.kernel, mesh=mesh,
        out_type=jax.ShapeDtypeStruct((n, dp), jnp.float32),
        scratch_types=[
            pltpu.VMEM((per_w,), jnp.int32),
            pltpu.VMEM((per_w, dp), jnp.float32),
            pltpu.SemaphoreType.DMA,
        ],
    )
    def k(table_hbm, idx_hbm, out_hbm, idx_v, rows_v, sem):
        wid = lax.axis_index("s") * nc + lax.axis_index("c")
        base = wid * per_w
        pltpu.sync_copy(idx_hbm.at[pl.ds(base, per_w)], idx_v)
        pltpu.async_copy(table_hbm.at[idx_v], rows_v, sem).wait()
        pltpu.sync_copy(rows_v, out_hbm.at[pl.ds(base, per_w)])

    return k(table_p, idx)[:, :_D]


def kernel(z, codebook):
    b, d, h, w = z.shape
    z_flat = jnp.transpose(z, (0, 2, 3, 1)).reshape(-1, d)
    z2 = jnp.sum(z_flat ** 2, axis=1, keepdims=True)
    c2 = jnp.sum(codebook ** 2, axis=1)[None, :]
    cb_bf = codebook.T.astype(jnp.bfloat16)
    lhs_bf = (2.0 * z_flat).astype(jnp.bfloat16)

    idx2d, loss11 = _dist_argmin(lhs_bf, cb_bf, z2, c2)
    indices = idx2d[:, 0]

    zq_flat = _sc_gather(codebook, indices)
    z_q = jnp.transpose(zq_flat.reshape(b, h, w, d), (0, 3, 1, 2))
    z_q = z + lax.stop_gradient(z_q - z)
    return (z_q, indices.reshape(b, h * w), loss11[0, 0])


# speedup vs baseline: 1.0575x; 1.0394x over previous
import functools

import jax
import jax.numpy as jnp
from jax import lax
from jax.experimental import pallas as pl
from jax.experimental.pallas import tpu as pltpu
from jax.experimental.pallas import tpu_sc as plsc

_K = 8192
_D = 32
_BETA = 0.25
_T = 512


_KBLK = 2048
_NK = _K // _KBLK


def _dist_argmin_body(lhs_ref, cb_ref, z2_ref, c2_ref, idx_ref, loss_ref):
    i = pl.program_id(0)

    iota = lax.broadcasted_iota(jnp.int32, (_T, _KBLK), 1)
    best_v = None
    best_i = None
    for k in range(_NK):
        mm = lax.dot_general(
            lhs_ref[...], cb_ref[:, k * _KBLK:(k + 1) * _KBLK],
            dimension_numbers=(((1,), (0,)), ((), ())),
            preferred_element_type=jnp.float32,
        )
        dist = (z2_ref[...] + c2_ref[:, k * _KBLK:(k + 1) * _KBLK]) - mm
        local_min = jnp.min(dist, axis=1, keepdims=True)
        masked = jnp.where(dist == local_min, iota, jnp.int32(2**31 - 1))
        local_idx = jnp.min(masked, axis=1, keepdims=True) + k * _KBLK
        if k == 0:
            best_v, best_i = local_min, local_idx
        else:
            upd = local_min < best_v
            best_v = jnp.where(upd, local_min, best_v)
            best_i = jnp.where(upd, local_idx, best_i)
        if k < _NK - 1:
            best_v = best_v.astype(jnp.bfloat16).astype(jnp.float32)

    idx_ref[...] = best_i
    partial = jnp.sum(best_v) * (_BETA / (_K * _D))

    @pl.when(i == 0)
    def _():
        loss_ref[...] = jnp.reshape(partial, (1, 1))

    @pl.when(i > 0)
    def _():
        loss_ref[...] = loss_ref[...] + jnp.reshape(partial, (1, 1))


def _dist_argmin(lhs_bf, c_t, z2, c2):
    n = lhs_bf.shape[0]
    grid = (n // _T,)
    return pl.pallas_call(
        _dist_argmin_body,
        grid=grid,
        in_specs=[
            pl.BlockSpec((_T, _D), lambda i: (i, 0)),
            pl.BlockSpec((_D, _K), lambda i: (0, 0)),
            pl.BlockSpec((_T, 1), lambda i: (i, 0)),
            pl.BlockSpec((1, _K), lambda i: (0, 0)),
        ],
        out_specs=[
            pl.BlockSpec((_T, 1), lambda i: (i, 0)),
            pl.BlockSpec((1, 1), lambda i: (0, 0)),
        ],
        out_shape=[
            jax.ShapeDtypeStruct((n, 1), jnp.int32),
            jax.ShapeDtypeStruct((1, 1), jnp.float32),
        ],
    )(lhs_bf, c_t, z2, c2)


def _sc_gather(table, idx):
    info = plsc.get_sparse_core_info()
    nc, ns = info.num_cores, info.num_subcores
    nw = nc * ns
    n = idx.shape[0]
    per_w = n // nw
    dp = 128
    table_p = jnp.pad(table, ((0, 0), (0, dp - _D)))
    mesh = plsc.VectorSubcoreMesh(core_axis_name="c", subcore_axis_name="s")

    @functools.partial(
        pl.kernel, mesh=mesh,
        out_type=jax.ShapeDtypeStruct((n, dp), jnp.float32),
        scratch_types=[
            pltpu.VMEM((per_w,), jnp.int32),
            pltpu.VMEM((per_w, dp), jnp.float32),
            pltpu.SemaphoreType.DMA,
        ],
    )
    def k(table_hbm, idx_hbm, out_hbm, idx_v, rows_v, sem):
        wid = lax.axis_index("s") * nc + lax.axis_index("c")
        base = wid * per_w
        pltpu.sync_copy(idx_hbm.at[pl.ds(base, per_w)], idx_v)
        pltpu.async_copy(table_hbm.at[idx_v], rows_v, sem).wait()
        pltpu.sync_copy(rows_v, out_hbm.at[pl.ds(base, per_w)])

    return k(table_p, idx)[:, :_D]


def kernel(z, codebook):
    b, d, h, w = z.shape
    z_flat = jnp.transpose(z, (0, 2, 3, 1)).reshape(-1, d)
    z2 = jnp.sum(z_flat ** 2, axis=1, keepdims=True)
    c2 = jnp.sum(codebook ** 2, axis=1)[None, :]
    cb_bf = codebook.T.astype(jnp.bfloat16)
    lhs_bf = (2.0 * z_flat).astype(jnp.bfloat16)

    idx2d, loss11 = _dist_argmin(lhs_bf, cb_bf, z2, c2)
    indices = idx2d[:, 0]

    zq_flat = _sc_gather(codebook, indices)
    z_q = jnp.transpose(zq_flat.reshape(b, h, w, d), (0, 3, 1, 2))
    z_q = z + lax.stop_gradient(z_q - z)
    return (z_q, indices.reshape(b, h * w), loss11[0, 0])


# speedup vs baseline: 1.0872x; 1.0280x over previous
import functools

import jax
import jax.numpy as jnp
from jax import lax
from jax.experimental import pallas as pl
from jax.experimental.pallas import tpu as pltpu
from jax.experimental.pallas import tpu_sc as plsc

_K = 8192
_D = 32
_BETA = 0.25
_T = 1024


_KBLK = 2048
_NK = _K // _KBLK


def _dist_argmin_body(lhs_ref, cb_ref, z2_ref, c2_ref, idx_ref, loss_ref):
    i = pl.program_id(0)

    iota = lax.broadcasted_iota(jnp.int32, (_T, _KBLK), 1)
    best_v = None
    best_i = None
    for k in range(_NK):
        mm = lax.dot_general(
            lhs_ref[...], cb_ref[:, k * _KBLK:(k + 1) * _KBLK],
            dimension_numbers=(((1,), (0,)), ((), ())),
            preferred_element_type=jnp.float32,
        )
        dist = (z2_ref[...] + c2_ref[:, k * _KBLK:(k + 1) * _KBLK]) - mm
        local_min = jnp.min(dist, axis=1, keepdims=True)
        masked = jnp.where(dist == local_min, iota, jnp.int32(2**31 - 1))
        local_idx = jnp.min(masked, axis=1, keepdims=True) + k * _KBLK
        if k == 0:
            best_v, best_i = local_min, local_idx
        else:
            upd = local_min < best_v
            best_v = jnp.where(upd, local_min, best_v)
            best_i = jnp.where(upd, local_idx, best_i)
        if k < _NK - 1:
            best_v = best_v.astype(jnp.bfloat16).astype(jnp.float32)

    idx_ref[...] = best_i
    partial = jnp.sum(best_v) * (_BETA / (_K * _D))

    @pl.when(i == 0)
    def _():
        loss_ref[...] = jnp.reshape(partial, (1, 1))

    @pl.when(i > 0)
    def _():
        loss_ref[...] = loss_ref[...] + jnp.reshape(partial, (1, 1))


def _dist_argmin(lhs_bf, c_t, z2, c2):
    n = lhs_bf.shape[0]
    grid = (n // _T,)
    return pl.pallas_call(
        _dist_argmin_body,
        grid=grid,
        in_specs=[
            pl.BlockSpec((_T, _D), lambda i: (i, 0)),
            pl.BlockSpec((_D, _K), lambda i: (0, 0)),
            pl.BlockSpec((_T, 1), lambda i: (i, 0)),
            pl.BlockSpec((1, _K), lambda i: (0, 0)),
        ],
        out_specs=[
            pl.BlockSpec((_T, 1), lambda i: (i, 0)),
            pl.BlockSpec((1, 1), lambda i: (0, 0)),
        ],
        out_shape=[
            jax.ShapeDtypeStruct((n, 1), jnp.int32),
            jax.ShapeDtypeStruct((1, 1), jnp.float32),
        ],
    )(lhs_bf, c_t, z2, c2)


def _sc_gather(table, idx):
    info = plsc.get_sparse_core_info()
    nc, ns = info.num_cores, info.num_subcores
    nw = nc * ns
    n = idx.shape[0]
    per_w = n // nw
    dp = 128
    table_p = jnp.pad(table, ((0, 0), (0, dp - _D)))
    mesh = plsc.VectorSubcoreMesh(core_axis_name="c", subcore_axis_name="s")

    @functools.partial(
        pl.kernel, mesh=mesh,
        out_type=jax.ShapeDtypeStruct((n, dp), jnp.float32),
        scratch_types=[
            pltpu.VMEM((per_w,), jnp.int32),
            pltpu.VMEM((per_w, dp), jnp.float32),
            pltpu.SemaphoreType.DMA,
        ],
    )
    def k(table_hbm, idx_hbm, out_hbm, idx_v, rows_v, sem):
        wid = lax.axis_index("s") * nc + lax.axis_index("c")
        base = wid * per_w
        pltpu.sync_copy(idx_hbm.at[pl.ds(base, per_w)], idx_v)
        pltpu.async_copy(table_hbm.at[idx_v], rows_v, sem).wait()
        pltpu.sync_copy(rows_v, out_hbm.at[pl.ds(base, per_w)])

    return k(table_p, idx)[:, :_D]


def kernel(z, codebook):
    b, d, h, w = z.shape
    z_flat = jnp.transpose(z, (0, 2, 3, 1)).reshape(-1, d)
    z2 = jnp.sum(z_flat ** 2, axis=1, keepdims=True)
    c2 = jnp.sum(codebook ** 2, axis=1)[None, :]
    cb_bf = codebook.T.astype(jnp.bfloat16)
    lhs_bf = (2.0 * z_flat).astype(jnp.bfloat16)

    idx2d, loss11 = _dist_argmin(lhs_bf, cb_bf, z2, c2)
    indices = idx2d[:, 0]

    zq_flat = _sc_gather(codebook, indices)
    z_q = jnp.transpose(zq_flat.reshape(b, h, w, d), (0, 3, 1, 2))
    z_q = z + lax.stop_gradient(z_q - z)
    return (z_q, indices.reshape(b, h * w), loss11[0, 0])
